# baseline (speedup 1.0000x reference)
import functools

import jax
import jax.numpy as jnp
from jax import lax
from jax.experimental import pallas as pl
from jax.experimental.pallas import tpu as pltpu
from jax.experimental.pallas import tpu_sc as plsc

SCALE = 2.5
TOPK = 2
M = 128
L = 16


def _gating_body(gwb_ref, xb_ref, scores_ref):
    logits = lax.dot_general(
        gwb_ref[...], xb_ref[...], (((1,), (1,)), ((), ())),
        preferred_element_type=jnp.float32)
    scores_ref[...] = 1.0 / (1.0 + jnp.exp(-logits))


def _dispatch_body(scores_hbm, bias_hbm, rt_hbm, te_hbm, pos_hbm, w_hbm,
                   s_v, b_v, i_v, w_v, p_v, rt_v, te_v, cnt_v, woff_v,
                   *, T, E, NTR, NTA):
    wid = lax.axis_index("s") * 2 + lax.axis_index("c")
    NCH = T // L
    SHB = NTR * M

    @pl.when(wid == 0)
    def _():
        pltpu.sync_copy(scores_hbm, s_v)
        pltpu.sync_copy(bias_hbm, b_v)
        lanes = lax.iota(jnp.int32, L)
        ones_i = jnp.ones((L,), jnp.int32)
        cnt_v[...] = jnp.zeros((L,), jnp.int32)

        def init0(c, _):
            rt_v[pl.ds(c * L, L)] = jnp.zeros((L,), jnp.int32)
            return 0
        lax.fori_loop(0, SHB // L, init0, 0)

        def init1(c, _):
            rt_v[pl.ds(SHB + c * L, L)] = lanes + c * L
            return 0
        lax.fori_loop(0, NCH, init1, 0)

        bvec = b_v[...]

        def routing(c, _):
            s = [s_v[e, pl.ds(c * L, L)] for e in range(E)]
            b = [s[e] + bvec[e] for e in range(E)]
            m1 = b[0]
            i1 = jnp.zeros((L,), jnp.int32)
            for e in range(1, E):
                gt = b[e] > m1
                m1 = jnp.where(gt, b[e], m1)
                i1 = jnp.where(gt, e, i1)
            m2 = jnp.full((L,), -1e30, jnp.float32)
            i2 = jnp.zeros((L,), jnp.int32)
            for e in range(E):
                ok = jnp.logical_and(i1 != e, b[e] > m2)
                m2 = jnp.where(ok, b[e], m2)
                i2 = jnp.where(ok, e, i2)
            w1 = jnp.zeros((L,), jnp.float32)
            w2 = jnp.zeros((L,), jnp.float32)
            for e in range(E):
                w1 = jnp.where(i1 == e, s[e], w1)
                w2 = jnp.where(i2 == e, s[e], w2)
            den = w1 + w2 + 1e-20
            w1 = w1 / den * SCALE
            w2 = w2 / den * SCALE
            i_v[0, pl.ds(c * L, L)] = i1
            i_v[1, pl.ds(c * L, L)] = i2
            w_v[0, pl.ds(c * L, L)] = w1
            w_v[1, pl.ds(c * L, L)] = w2
            plsc.addupdate_scatter(cnt_v, [i1], ones_i)
            plsc.addupdate_scatter(cnt_v, [i2], ones_i)
            return 0
        lax.fori_loop(0, NCH, routing, 0)

        cnt = cnt_v[...]
        pc = ((cnt + (M - 1)) >> 7) << 7
        cum = plsc.cumsum(pc)
        woff_v[...] = cum - pc
        for jc in range(NTA // L + 1):
            jv = (lanes + jc * L) * M
            tej = jnp.zeros((L,), jnp.int32)
            for e in range(E):
                tej += (jv >= cum[e]).astype(jnp.int32)
            te_v[pl.ds(jc * L, L)] = tej

        def place(c, _):
            toks = lanes + c * L
            for k in range(TOPK):
                ev = i_v[k, pl.ds(c * L, L)]
                bases = plsc.load_gather(woff_v, [ev])
                rank = jnp.zeros((L,), jnp.int32)
                for e in range(E):
                    mk = ev == e
                    cs = plsc.cumsum(mk.astype(jnp.int32))
                    rank = jnp.where(mk, cs - 1, rank)
                posv = bases + rank
                plsc.addupdate_scatter(woff_v, [ev], ones_i)
                plsc.store_scatter(rt_v, [posv], toks)
                p_v[k, pl.ds(c * L, L)] = posv
            return 0
        lax.fori_loop(0, NCH, place, 0)

        pltpu.sync_copy(rt_v, rt_hbm)
        pltpu.sync_copy(te_v, te_hbm)
        pltpu.sync_copy(p_v, pos_hbm)
        pltpu.sync_copy(w_v, w_hbm)


def _gather_body(x_hbm, rt_hbm, xs_hbm, idx_v, buf0, buf1, gsem0, gsem1,
                 osem0, osem1, *, NR, RPW, CH):
    wid = lax.axis_index("s") * 2 + lax.axis_index("c")
    base = wid * RPW
    pltpu.sync_copy(rt_hbm.at[pl.ds(base, RPW)], idx_v)
    bufs = (buf0, buf1)
    gsems = (gsem0, gsem1)
    osems = (osem0, osem1)
    nch = RPW // CH
    gcp = [None, None]
    ocp = [None, None]

    def start_fetch(ch, sl):
        gcp[sl] = pltpu.async_copy(
            x_hbm.at[idx_v.at[pl.ds(ch * CH, CH)]], bufs[sl], gsems[sl])

    start_fetch(0, 0)
    for ch in range(nch):
        sl = ch & 1
        if ch + 1 < nch:
            if ocp[1 - sl] is not None:
                ocp[1 - sl].wait()
            start_fetch(ch + 1, 1 - sl)
        gcp[sl].wait()
        ocp[sl] = pltpu.async_copy(
            bufs[sl], xs_hbm.at[pl.ds(base + ch * CH, CH)], osems[sl])
    for sl in range(2):
        if ocp[sl] is not None:
            ocp[sl].wait()


def _gmm_body(te_ref, xs_ref, wgu_ref, wdn_ref, eo_ref, *, FF):
    xw = xs_ref[...]
    xe = lax.bitcast_convert_type(xw << 16, jnp.float32)
    xo = lax.bitcast_convert_type(xw & jnp.int32(-65536), jnp.float32)
    xb = jnp.concatenate([xe, xo], axis=1).astype(jnp.bfloat16)
    gu = lax.dot_general(
        xb, wgu_ref[0], (((1,), (0,)), ((), ())),
        preferred_element_type=jnp.float32)
    g = gu[:, :FF]
    u = gu[:, FF:]
    act = (g * (1.0 / (1.0 + jnp.exp(-g))) * u).astype(jnp.bfloat16)
    eo_ref[...] = lax.dot_general(
        act, wdn_ref[0], (((1,), (0,)), ((), ())),
        preferred_element_type=jnp.float32)


def _combine_body(eo_hbm, pos_hbm, w_hbm, out_hbm, p_v, w_v, g0a, g0b,
                  g1a, g1b, sha, shb, sg0a, sg0b, sg1a, sg1b, ssha, sshb,
                  soa, sob, *, T, H, SHB, TPW, CH):
    wid = lax.axis_index("s") * 2 + lax.axis_index("c")
    base = wid * TPW
    pltpu.sync_copy(pos_hbm.at[0, pl.ds(base, TPW)], p_v.at[0])
    pltpu.sync_copy(pos_hbm.at[1, pl.ds(base, TPW)], p_v.at[1])
    pltpu.sync_copy(w_hbm.at[0, pl.ds(base, TPW)], w_v.at[0])
    pltpu.sync_copy(w_hbm.at[1, pl.ds(base, TPW)], w_v.at[1])
    nch = TPW // CH
    nv = H // L
    g0s = (g0a, g0b)
    g1s = (g1a, g1b)
    shs = (sha, shb)
    gsems = ((sg0a, sg1a, ssha), (sg0b, sg1b, sshb))
    osems = (soa, sob)
    fcp = [None, None]
    ocp = [None, None]

    def start_fetch(ch, sl):
        s0, s1, s2 = gsems[sl]
        fcp[sl] = (
            pltpu.async_copy(
                eo_hbm.at[p_v.at[0, pl.ds(ch * CH, CH)]], g0s[sl], s0),
            pltpu.async_copy(
                eo_hbm.at[p_v.at[1, pl.ds(ch * CH, CH)]], g1s[sl], s1),
            pltpu.async_copy(
                eo_hbm.at[pl.ds(SHB + base + ch * CH, CH)], shs[sl], s2),
        )

    start_fetch(0, 0)
    for ch in range(nch):
        sl = ch & 1
        if ch + 1 < nch:
            if ocp[1 - sl] is not None:
                ocp[1 - sl].wait()
            start_fetch(ch + 1, 1 - sl)
        for cp in fcp[sl]:
            cp.wait()
        g0 = g0s[sl]
        g1 = g1s[sl]
        sh = shs[sl]
        w0vec = w_v[0, pl.ds((ch // 2) * L, L)]
        w1vec = w_v[1, pl.ds((ch // 2) * L, L)]
        for r in range(CH):
            w0 = w0vec[(ch % 2) * CH + r]
            w1 = w1vec[(ch % 2) * CH + r]

            def row(v, _):
                sh[r, pl.ds(v * L, L)] = (
                    sh[r, pl.ds(v * L, L)]
                    + g0[r, pl.ds(v * L, L)] * w0
                    + g1[r, pl.ds(v * L, L)] * w1)
                return 0
            lax.fori_loop(0, nv, row, 0)
        ocp[sl] = pltpu.async_copy(
            sh, out_hbm.at[pl.ds(base + ch * CH, CH)], osems[sl])
    for sl in range(2):
        if ocp[sl] is not None:
            ocp[sl].wait()


def kernel(x, gate_w, e_score_correction_bias, w_gate_up, w_down,
           ws_gate_up, ws_down):
    T, H = x.shape
    E = gate_w.shape[0]
    FF = w_down.shape[1]
    NTR = (T * TOPK) // M + E
    SHB = NTR * M
    NR = SHB + T
    NTA = NTR + T // M
    NTE = ((NTA // L) + 1) * L

    xb = x.astype(jnp.bfloat16)
    xp = lax.bitcast_convert_type(
        xb.reshape(T, H // 2, 2), jnp.int32)
    gwb = gate_w.astype(jnp.bfloat16)
    bias16 = jnp.zeros((L,), jnp.float32).at[:E].set(e_score_correction_bias)
    wgu_all = jnp.concatenate(
        [w_gate_up, ws_gate_up[None]], axis=0)
    wgu_all = jnp.concatenate(
        [wgu_all[:, 0::2, :], wgu_all[:, 1::2, :]], axis=1
    ).astype(jnp.bfloat16)
    wdn_all = jnp.concatenate(
        [w_down, ws_down[None]], axis=0).astype(jnp.bfloat16)

    scores = pl.pallas_call(
        _gating_body,
        grid=(1,),
        in_specs=[
            pl.BlockSpec((E, H), lambda i: (0, 0)),
            pl.BlockSpec((T, H), lambda i: (0, 0)),
        ],
        out_specs=pl.BlockSpec((E, T), lambda i: (0, 0)),
        out_shape=jax.ShapeDtypeStruct((E, T), jnp.float32),
    )(gwb, xb)

    mesh = plsc.VectorSubcoreMesh(core_axis_name="c", subcore_axis_name="s")
    rt, te, pos, w01 = pl.kernel(
        functools.partial(_dispatch_body, T=T, E=E, NTR=NTR, NTA=NTA),
        out_type=[
            jax.ShapeDtypeStruct((NR,), jnp.int32),
            jax.ShapeDtypeStruct((NTE,), jnp.int32),
            jax.ShapeDtypeStruct((TOPK, T), jnp.int32),
            jax.ShapeDtypeStruct((TOPK, T), jnp.float32),
        ],
        mesh=mesh,
        compiler_params=pltpu.CompilerParams(needs_layout_passes=False),
        scratch_types=[
            pltpu.VMEM((E, T), jnp.float32),
            pltpu.VMEM((L,), jnp.float32),
            pltpu.VMEM((TOPK, T), jnp.int32),
            pltpu.VMEM((TOPK, T), jnp.float32),
            pltpu.VMEM((TOPK, T), jnp.int32),
            pltpu.VMEM((NR,), jnp.int32),
            pltpu.VMEM((NTE,), jnp.int32),
            pltpu.VMEM((L,), jnp.int32),
            pltpu.VMEM((L,), jnp.int32),
        ],
    )(scores, bias16)

    RPW = NR // 32
    CH = 32
    HP = H // 2
    xs = pl.kernel(
        functools.partial(_gather_body, NR=NR, RPW=RPW, CH=CH),
        out_type=jax.ShapeDtypeStruct((NR, HP), jnp.int32),
        mesh=mesh,
        compiler_params=pltpu.CompilerParams(needs_layout_passes=False),
        scratch_types=[
            pltpu.VMEM((RPW,), jnp.int32),
            pltpu.VMEM((CH, HP), jnp.int32),
            pltpu.VMEM((CH, HP), jnp.int32),
            pltpu.SemaphoreType.DMA,
            pltpu.SemaphoreType.DMA,
            pltpu.SemaphoreType.DMA,
            pltpu.SemaphoreType.DMA,
        ],
    )(xp, rt)

    eo = pl.pallas_call(
        functools.partial(_gmm_body, FF=FF),
        grid_spec=pltpu.PrefetchScalarGridSpec(
            num_scalar_prefetch=1,
            grid=(NTA,),
            in_specs=[
                pl.BlockSpec((M, H // 2), lambda j, te_r: (j, 0)),
                pl.BlockSpec((1, H, 2 * FF), lambda j, te_r: (te_r[j], 0, 0)),
                pl.BlockSpec((1, FF, H), lambda j, te_r: (te_r[j], 0, 0)),
            ],
            out_specs=pl.BlockSpec((M, H), lambda j, te_r: (j, 0)),
        ),
        out_shape=jax.ShapeDtypeStruct((NR, H), jnp.float32),
        compiler_params=pltpu.CompilerParams(
            dimension_semantics=("arbitrary",)),
    )(te, xs, wgu_all, wdn_all)

    TPW = T // 32
    CHE = 8
    out = pl.kernel(
        functools.partial(_combine_body, T=T, H=H, SHB=SHB, TPW=TPW, CH=CHE),
        out_type=jax.ShapeDtypeStruct((T, H), jnp.float32),
        mesh=mesh,
        compiler_params=pltpu.CompilerParams(needs_layout_passes=False),
        scratch_types=(
            [pltpu.VMEM((TOPK, TPW), jnp.int32),
             pltpu.VMEM((TOPK, TPW), jnp.float32)]
            + [pltpu.VMEM((CHE, H), jnp.float32) for _ in range(6)]
            + [pltpu.SemaphoreType.DMA for _ in range(8)]
        ),
    )(eo, pos, w01)
    return out


# speedup vs baseline: 2.2336x; 2.2336x over previous
import functools

import jax
import jax.numpy as jnp
from jax import lax
from jax.experimental import pallas as pl
from jax.experimental.pallas import tpu as pltpu
from jax.experimental.pallas import tpu_sc as plsc

SCALE = 2.5
TOPK = 2
M = 128
L = 16


def _gating_body(gwb_ref, xb_ref, scores_ref):
    logits = lax.dot_general(
        gwb_ref[...], xb_ref[...], (((1,), (1,)), ((), ())),
        preferred_element_type=jnp.float32)
    scores_ref[...] = 1.0 / (1.0 + jnp.exp(-logits))


def _dispatch_body(scores_hbm, bias_hbm, rt_hbm, te_hbm, pos_hbm, w_hbm,
                   s_v, b_v, i_v, w_v, p_v, rt_v, te_v, cnt_v, woff_v,
                   *, T, E, NTR, NTA):
    wid = lax.axis_index("s") * 2 + lax.axis_index("c")
    NCH = T // L
    SHB = NTR * M

    @pl.when(wid == 0)
    def _():
        pltpu.sync_copy(scores_hbm, s_v)
        pltpu.sync_copy(bias_hbm, b_v)
        lanes = lax.iota(jnp.int32, L)
        ones_i = jnp.ones((L,), jnp.int32)
        cnt_v[...] = jnp.zeros((L,), jnp.int32)

        def init0(c, _):
            rt_v[pl.ds(c * L, L)] = jnp.zeros((L,), jnp.int32)
            return 0
        lax.fori_loop(0, SHB // L, init0, 0)

        def init1(c, _):
            rt_v[pl.ds(SHB + c * L, L)] = lanes + c * L
            return 0
        lax.fori_loop(0, NCH, init1, 0)

        bvec = b_v[...]

        def routing(c, _):
            s = [s_v[e, pl.ds(c * L, L)] for e in range(E)]
            b = [s[e] + bvec[e] for e in range(E)]
            m1 = b[0]
            i1 = jnp.zeros((L,), jnp.int32)
            for e in range(1, E):
                gt = b[e] > m1
                m1 = jnp.where(gt, b[e], m1)
                i1 = jnp.where(gt, e, i1)
            m2 = jnp.full((L,), -1e30, jnp.float32)
            i2 = jnp.zeros((L,), jnp.int32)
            for e in range(E):
                ok = jnp.logical_and(i1 != e, b[e] > m2)
                m2 = jnp.where(ok, b[e], m2)
                i2 = jnp.where(ok, e, i2)
            w1 = jnp.zeros((L,), jnp.float32)
            w2 = jnp.zeros((L,), jnp.float32)
            for e in range(E):
                w1 = jnp.where(i1 == e, s[e], w1)
                w2 = jnp.where(i2 == e, s[e], w2)
            den = w1 + w2 + 1e-20
            w1 = w1 / den * SCALE
            w2 = w2 / den * SCALE
            i_v[0, pl.ds(c * L, L)] = i1
            i_v[1, pl.ds(c * L, L)] = i2
            w_v[0, pl.ds(c * L, L)] = w1
            w_v[1, pl.ds(c * L, L)] = w2
            plsc.addupdate_scatter(cnt_v, [i1], ones_i)
            plsc.addupdate_scatter(cnt_v, [i2], ones_i)
            return 0
        lax.fori_loop(0, NCH, routing, 0)

        cnt = cnt_v[...]
        pc = ((cnt + (M - 1)) >> 7) << 7
        cum = plsc.cumsum(pc)
        woff_v[...] = cum - pc
        for jc in range(NTA // L + 1):
            jv = (lanes + jc * L) * M
            tej = jnp.zeros((L,), jnp.int32)
            for e in range(E):
                tej += (jv >= cum[e]).astype(jnp.int32)
            te_v[pl.ds(jc * L, L)] = tej

        def place(c, _):
            toks = lanes + c * L
            for k in range(TOPK):
                ev = i_v[k, pl.ds(c * L, L)]
                bases = plsc.load_gather(woff_v, [ev])
                rank = jnp.zeros((L,), jnp.int32)
                for e in range(E):
                    mk = ev == e
                    cs = plsc.cumsum(mk.astype(jnp.int32))
                    rank = jnp.where(mk, cs - 1, rank)
                posv = bases + rank
                plsc.addupdate_scatter(woff_v, [ev], ones_i)
                plsc.store_scatter(rt_v, [posv], toks)
                p_v[k, pl.ds(c * L, L)] = posv
            return 0
        lax.fori_loop(0, NCH, place, 0)

        pltpu.sync_copy(rt_v, rt_hbm)
        pltpu.sync_copy(te_v, te_hbm)
        pltpu.sync_copy(p_v, pos_hbm)
        pltpu.sync_copy(w_v, w_hbm)


def _gather_body(x_hbm, rt_hbm, xs_hbm, idx_v, buf0, buf1, gsem0, gsem1,
                 osem0, osem1, *, NR, RPW, CH):
    wid = lax.axis_index("s") * 2 + lax.axis_index("c")
    base = wid * RPW
    pltpu.sync_copy(rt_hbm.at[pl.ds(base, RPW)], idx_v)
    bufs = (buf0, buf1)
    gsems = (gsem0, gsem1)
    osems = (osem0, osem1)
    nch = RPW // CH
    gcp = [None, None]
    ocp = [None, None]

    def start_fetch(ch, sl):
        gcp[sl] = pltpu.async_copy(
            x_hbm.at[idx_v.at[pl.ds(ch * CH, CH)]], bufs[sl], gsems[sl])

    start_fetch(0, 0)
    for ch in range(nch):
        sl = ch & 1
        if ch + 1 < nch:
            if ocp[1 - sl] is not None:
                ocp[1 - sl].wait()
            start_fetch(ch + 1, 1 - sl)
        gcp[sl].wait()
        ocp[sl] = pltpu.async_copy(
            bufs[sl], xs_hbm.at[pl.ds(base + ch * CH, CH)], osems[sl])
    for sl in range(2):
        if ocp[sl] is not None:
            ocp[sl].wait()


def _gmm_body(te_ref, xs_ref, wgu_ref, wdn_ref, eo_ref, *, FF):
    xb = xs_ref[...].astype(jnp.bfloat16)
    gu = lax.dot_general(
        xb, wgu_ref[0], (((1,), (0,)), ((), ())),
        preferred_element_type=jnp.float32)
    g = gu[:, :FF]
    u = gu[:, FF:]
    act = (g * (1.0 / (1.0 + jnp.exp(-g))) * u).astype(jnp.bfloat16)
    eo_ref[...] = lax.dot_general(
        act, wdn_ref[0], (((1,), (0,)), ((), ())),
        preferred_element_type=jnp.float32)


def _combine_body(eo_hbm, pos_hbm, w_hbm, out_hbm, p_v, w_v, g0a, g0b,
                  g1a, g1b, sha, shb, sg0a, sg0b, sg1a, sg1b, ssha, sshb,
                  soa, sob, *, T, H, SHB, TPW, CH):
    wid = lax.axis_index("s") * 2 + lax.axis_index("c")
    base = wid * TPW
    pltpu.sync_copy(pos_hbm.at[0, pl.ds(base, TPW)], p_v.at[0])
    pltpu.sync_copy(pos_hbm.at[1, pl.ds(base, TPW)], p_v.at[1])
    pltpu.sync_copy(w_hbm.at[0, pl.ds(base, TPW)], w_v.at[0])
    pltpu.sync_copy(w_hbm.at[1, pl.ds(base, TPW)], w_v.at[1])
    nch = TPW // CH
    nv = H // L
    g0s = (g0a, g0b)
    g1s = (g1a, g1b)
    shs = (sha, shb)
    gsems = ((sg0a, sg1a, ssha), (sg0b, sg1b, sshb))
    osems = (soa, sob)
    fcp = [None, None]
    ocp = [None, None]

    def start_fetch(ch, sl):
        s0, s1, s2 = gsems[sl]
        fcp[sl] = (
            pltpu.async_copy(
                eo_hbm.at[p_v.at[0, pl.ds(ch * CH, CH)]], g0s[sl], s0),
            pltpu.async_copy(
                eo_hbm.at[p_v.at[1, pl.ds(ch * CH, CH)]], g1s[sl], s1),
            pltpu.async_copy(
                eo_hbm.at[pl.ds(SHB + base + ch * CH, CH)], shs[sl], s2),
        )

    start_fetch(0, 0)
    for ch in range(nch):
        sl = ch & 1
        if ch + 1 < nch:
            if ocp[1 - sl] is not None:
                ocp[1 - sl].wait()
            start_fetch(ch + 1, 1 - sl)
        for cp in fcp[sl]:
            cp.wait()
        g0 = g0s[sl]
        g1 = g1s[sl]
        sh = shs[sl]
        w0vec = w_v[0, pl.ds((ch // 2) * L, L)]
        w1vec = w_v[1, pl.ds((ch // 2) * L, L)]
        for r in range(CH):
            w0 = w0vec[(ch % 2) * CH + r]
            w1 = w1vec[(ch % 2) * CH + r]

            def row(v, _):
                sh[r, pl.ds(v * L, L)] = (
                    sh[r, pl.ds(v * L, L)]
                    + g0[r, pl.ds(v * L, L)] * w0
                    + g1[r, pl.ds(v * L, L)] * w1)
                return 0
            lax.fori_loop(0, nv, row, 0)
        ocp[sl] = pltpu.async_copy(
            sh, out_hbm.at[pl.ds(base + ch * CH, CH)], osems[sl])
    for sl in range(2):
        if ocp[sl] is not None:
            ocp[sl].wait()


def kernel(x, gate_w, e_score_correction_bias, w_gate_up, w_down,
           ws_gate_up, ws_down):
    T, H = x.shape
    E = gate_w.shape[0]
    FF = w_down.shape[1]
    NTR = (T * TOPK) // M + E
    SHB = NTR * M
    NR = SHB + T
    NTA = NTR + T // M
    NTE = ((NTA // L) + 1) * L

    xb = x.astype(jnp.bfloat16)
    gwb = gate_w.astype(jnp.bfloat16)
    bias16 = jnp.zeros((L,), jnp.float32).at[:E].set(e_score_correction_bias)
    wgu_all = jnp.concatenate(
        [w_gate_up, ws_gate_up[None]], axis=0).astype(jnp.bfloat16)
    wdn_all = jnp.concatenate(
        [w_down, ws_down[None]], axis=0).astype(jnp.bfloat16)

    scores = pl.pallas_call(
        _gating_body,
        grid=(1,),
        in_specs=[
            pl.BlockSpec((E, H), lambda i: (0, 0)),
            pl.BlockSpec((T, H), lambda i: (0, 0)),
        ],
        out_specs=pl.BlockSpec((E, T), lambda i: (0, 0)),
        out_shape=jax.ShapeDtypeStruct((E, T), jnp.float32),
    )(gwb, xb)

    mesh = plsc.VectorSubcoreMesh(core_axis_name="c", subcore_axis_name="s")
    rt, te, pos, w01 = pl.kernel(
        functools.partial(_dispatch_body, T=T, E=E, NTR=NTR, NTA=NTA),
        out_type=[
            jax.ShapeDtypeStruct((NR,), jnp.int32),
            jax.ShapeDtypeStruct((NTE,), jnp.int32),
            jax.ShapeDtypeStruct((TOPK, T), jnp.int32),
            jax.ShapeDtypeStruct((TOPK, T), jnp.float32),
        ],
        mesh=mesh,
        compiler_params=pltpu.CompilerParams(needs_layout_passes=False),
        scratch_types=[
            pltpu.VMEM((E, T), jnp.float32),
            pltpu.VMEM((L,), jnp.float32),
            pltpu.VMEM((TOPK, T), jnp.int32),
            pltpu.VMEM((TOPK, T), jnp.float32),
            pltpu.VMEM((TOPK, T), jnp.int32),
            pltpu.VMEM((NR,), jnp.int32),
            pltpu.VMEM((NTE,), jnp.int32),
            pltpu.VMEM((L,), jnp.int32),
            pltpu.VMEM((L,), jnp.int32),
        ],
    )(scores, bias16)

    RPW = NR // 32
    CH = 16
    xs = pl.kernel(
        functools.partial(_gather_body, NR=NR, RPW=RPW, CH=CH),
        out_type=jax.ShapeDtypeStruct((NR, H), jnp.float32),
        mesh=mesh,
        compiler_params=pltpu.CompilerParams(needs_layout_passes=False),
        scratch_types=[
            pltpu.VMEM((RPW,), jnp.int32),
            pltpu.VMEM((CH, H), jnp.float32),
            pltpu.VMEM((CH, H), jnp.float32),
            pltpu.SemaphoreType.DMA,
            pltpu.SemaphoreType.DMA,
            pltpu.SemaphoreType.DMA,
            pltpu.SemaphoreType.DMA,
        ],
    )(x, rt)

    eo = pl.pallas_call(
        functools.partial(_gmm_body, FF=FF),
        grid_spec=pltpu.PrefetchScalarGridSpec(
            num_scalar_prefetch=1,
            grid=(NTA,),
            in_specs=[
                pl.BlockSpec((M, H), lambda j, te_r: (j, 0)),
                pl.BlockSpec((1, H, 2 * FF), lambda j, te_r: (te_r[j], 0, 0)),
                pl.BlockSpec((1, FF, H), lambda j, te_r: (te_r[j], 0, 0)),
            ],
            out_specs=pl.BlockSpec((M, H), lambda j, te_r: (j, 0)),
        ),
        out_shape=jax.ShapeDtypeStruct((NR, H), jnp.float32),
        compiler_params=pltpu.CompilerParams(
            dimension_semantics=("arbitrary",)),
    )(te, xs, wgu_all, wdn_all)

    TPW = T // 32
    CHE = 8
    out = pl.kernel(
        functools.partial(_combine_body, T=T, H=H, SHB=SHB, TPW=TPW, CH=CHE),
        out_type=jax.ShapeDtypeStruct((T, H), jnp.float32),
        mesh=mesh,
        compiler_params=pltpu.CompilerParams(needs_layout_passes=False),
        scratch_types=(
            [pltpu.VMEM((TOPK, TPW), jnp.int32),
             pltpu.VMEM((TOPK, TPW), jnp.float32)]
            + [pltpu.VMEM((CHE, H), jnp.float32) for _ in range(6)]
            + [pltpu.SemaphoreType.DMA for _ in range(8)]
        ),
    )(eo, pos, w01)
    return out


# speedup vs baseline: 2.6288x; 1.1769x over previous
import functools

import jax
import jax.numpy as jnp
from jax import lax
from jax.experimental import pallas as pl
from jax.experimental.pallas import tpu as pltpu
from jax.experimental.pallas import tpu_sc as plsc

SCALE = 2.5
TOPK = 2
M = 128
L = 16


def _gating_body(gwb_ref, xb_ref, scores_ref):
    logits = lax.dot_general(
        gwb_ref[...], xb_ref[...], (((1,), (1,)), ((), ())),
        preferred_element_type=jnp.float32)
    scores_ref[...] = 1.0 / (1.0 + jnp.exp(-logits))


def _dispatch_body(scores_hbm, bias_hbm, rt_hbm, te_hbm, pos_hbm, w_hbm,
                   s_v, b_v, i_v, w_v, p_v, rt_v, te_v, cnt_v, woff_v,
                   *, T, E, NTR, NTA):
    wid = lax.axis_index("s") * 2 + lax.axis_index("c")
    NCH = T // L
    SHB = NTR * M

    @pl.when(wid == 0)
    def _():
        pltpu.sync_copy(scores_hbm, s_v)
        pltpu.sync_copy(bias_hbm, b_v)
        lanes = lax.iota(jnp.int32, L)
        ones_i = jnp.ones((L,), jnp.int32)
        cnt_v[...] = jnp.zeros((L,), jnp.int32)

        def init0(c, _):
            rt_v[pl.ds(c * L, L)] = jnp.zeros((L,), jnp.int32)
            return 0
        lax.fori_loop(0, SHB // L, init0, 0)

        bvec = b_v[...]

        def routing(c, _):
            s = [s_v[e, pl.ds(c * L, L)] for e in range(E)]
            b = [s[e] + bvec[e] for e in range(E)]
            m1 = b[0]
            i1 = jnp.zeros((L,), jnp.int32)
            for e in range(1, E):
                gt = b[e] > m1
                m1 = jnp.where(gt, b[e], m1)
                i1 = jnp.where(gt, e, i1)
            m2 = jnp.full((L,), -1e30, jnp.float32)
            i2 = jnp.zeros((L,), jnp.int32)
            for e in range(E):
                ok = jnp.logical_and(i1 != e, b[e] > m2)
                m2 = jnp.where(ok, b[e], m2)
                i2 = jnp.where(ok, e, i2)
            w1 = jnp.zeros((L,), jnp.float32)
            w2 = jnp.zeros((L,), jnp.float32)
            for e in range(E):
                w1 = jnp.where(i1 == e, s[e], w1)
                w2 = jnp.where(i2 == e, s[e], w2)
            den = w1 + w2 + 1e-20
            w1 = w1 / den * SCALE
            w2 = w2 / den * SCALE
            i_v[0, pl.ds(c * L, L)] = i1
            i_v[1, pl.ds(c * L, L)] = i2
            w_v[0, pl.ds(c * L, L)] = w1
            w_v[1, pl.ds(c * L, L)] = w2
            plsc.addupdate_scatter(cnt_v, [i1], ones_i)
            plsc.addupdate_scatter(cnt_v, [i2], ones_i)
            return 0
        lax.fori_loop(0, NCH, routing, 0)

        cnt = cnt_v[...]
        pc = ((cnt + (M - 1)) >> 7) << 7
        cum = plsc.cumsum(pc)
        woff_v[...] = cum - pc
        for jc in range(NTA // L + 1):
            jv = (lanes + jc * L) * M
            tej = jnp.zeros((L,), jnp.int32)
            for e in range(E):
                tej += (jv >= cum[e]).astype(jnp.int32)
            te_v[pl.ds(jc * L, L)] = jnp.minimum(tej, E - 1)

        def place(c, _):
            toks = lanes + c * L
            for k in range(TOPK):
                ev = i_v[k, pl.ds(c * L, L)]
                bases = plsc.load_gather(woff_v, [ev])
                rank = jnp.zeros((L,), jnp.int32)
                for e in range(E):
                    mk = ev == e
                    cs = plsc.cumsum(mk.astype(jnp.int32))
                    rank = jnp.where(mk, cs - 1, rank)
                posv = bases + rank
                plsc.addupdate_scatter(woff_v, [ev], ones_i)
                plsc.store_scatter(rt_v, [posv], toks)
                p_v[k, pl.ds(c * L, L)] = posv
            return 0
        lax.fori_loop(0, NCH, place, 0)

        pltpu.sync_copy(rt_v, rt_hbm)
        pltpu.sync_copy(te_v, te_hbm)
        pltpu.sync_copy(p_v, pos_hbm)
        pltpu.sync_copy(w_v, w_hbm)


def _gather_body(x_hbm, rt_hbm, xs_hbm, idx_v, buf0, buf1, gsem0, gsem1,
                 osem0, osem1, *, NR, RPW, CH):
    wid = lax.axis_index("s") * 2 + lax.axis_index("c")
    base = wid * RPW
    pltpu.sync_copy(rt_hbm.at[pl.ds(base, RPW)], idx_v)
    bufs = (buf0, buf1)
    gsems = (gsem0, gsem1)
    osems = (osem0, osem1)
    nch = RPW // CH
    gcp = [None, None]
    ocp = [None, None]

    def start_fetch(ch, sl):
        gcp[sl] = pltpu.async_copy(
            x_hbm.at[idx_v.at[pl.ds(ch * CH, CH)]], bufs[sl], gsems[sl])

    start_fetch(0, 0)
    for ch in range(nch):
        sl = ch & 1
        if ch + 1 < nch:
            if ocp[1 - sl] is not None:
                ocp[1 - sl].wait()
            start_fetch(ch + 1, 1 - sl)
        gcp[sl].wait()
        ocp[sl] = pltpu.async_copy(
            bufs[sl], xs_hbm.at[pl.ds(base + ch * CH, CH)], osems[sl])
    for sl in range(2):
        if ocp[sl] is not None:
            ocp[sl].wait()


def _gmm_body(te_ref, xs_ref, wgu_ref, wdn_ref, eo_ref, *, FF):
    xb = xs_ref[...].astype(jnp.bfloat16)
    gu = lax.dot_general(
        xb, wgu_ref[0], (((1,), (0,)), ((), ())),
        preferred_element_type=jnp.float32)
    g = gu[:, :FF]
    u = gu[:, FF:]
    act = (g * (1.0 / (1.0 + jnp.exp(-g))) * u).astype(jnp.bfloat16)
    eo_ref[...] = lax.dot_general(
        act, wdn_ref[0], (((1,), (0,)), ((), ())),
        preferred_element_type=jnp.float32)


def _shared_body(x_ref, wgu_ref, wdn_ref, eo_ref, *, FF):
    xb = x_ref[...].astype(jnp.bfloat16)
    gu = lax.dot_general(
        xb, wgu_ref[...], (((1,), (0,)), ((), ())),
        preferred_element_type=jnp.float32)
    g = gu[:, :FF]
    u = gu[:, FF:]
    act = (g * (1.0 / (1.0 + jnp.exp(-g))) * u).astype(jnp.bfloat16)
    eo_ref[...] = lax.dot_general(
        act, wdn_ref[...], (((1,), (0,)), ((), ())),
        preferred_element_type=jnp.float32)


def _combine_body(eo_hbm, eos_hbm, pos_hbm, w_hbm, out_hbm, p_v, w_v,
                  g0a, g0b, g1a, g1b, sha, shb, sg0a, sg0b, sg1a, sg1b,
                  ssha, sshb, soa, sob, *, T, H, TPW, CH):
    wid = lax.axis_index("s") * 2 + lax.axis_index("c")
    base = wid * TPW
    pltpu.sync_copy(pos_hbm.at[0, pl.ds(base, TPW)], p_v.at[0])
    pltpu.sync_copy(pos_hbm.at[1, pl.ds(base, TPW)], p_v.at[1])
    pltpu.sync_copy(w_hbm.at[0, pl.ds(base, TPW)], w_v.at[0])
    pltpu.sync_copy(w_hbm.at[1, pl.ds(base, TPW)], w_v.at[1])
    nch = TPW // CH
    nv = H // L
    g0s = (g0a, g0b)
    g1s = (g1a, g1b)
    shs = (sha, shb)
    gsems = ((sg0a, sg1a, ssha), (sg0b, sg1b, sshb))
    osems = (soa, sob)
    fcp = [None, None]
    ocp = [None, None]

    def start_fetch(ch, sl):
        s0, s1, s2 = gsems[sl]
        fcp[sl] = (
            pltpu.async_copy(
                eo_hbm.at[p_v.at[0, pl.ds(ch * CH, CH)]], g0s[sl], s0),
            pltpu.async_copy(
                eo_hbm.at[p_v.at[1, pl.ds(ch * CH, CH)]], g1s[sl], s1),
            pltpu.async_copy(
                eos_hbm.at[pl.ds(base + ch * CH, CH)], shs[sl], s2),
        )

    start_fetch(0, 0)
    for ch in range(nch):
        sl = ch & 1
        if ch + 1 < nch:
            if ocp[1 - sl] is not None:
                ocp[1 - sl].wait()
            start_fetch(ch + 1, 1 - sl)
        for cp in fcp[sl]:
            cp.wait()
        g0 = g0s[sl]
        g1 = g1s[sl]
        sh = shs[sl]
        w0vec = w_v[0, pl.ds((ch // 2) * L, L)]
        w1vec = w_v[1, pl.ds((ch // 2) * L, L)]
        for r in range(CH):
            w0 = w0vec[(ch % 2) * CH + r]
            w1 = w1vec[(ch % 2) * CH + r]

            def row(v, _):
                sh[r, pl.ds(v * L, L)] = (
                    sh[r, pl.ds(v * L, L)]
                    + g0[r, pl.ds(v * L, L)] * w0
                    + g1[r, pl.ds(v * L, L)] * w1)
                return 0
            lax.fori_loop(0, nv, row, 0)
        ocp[sl] = pltpu.async_copy(
            sh, out_hbm.at[pl.ds(base + ch * CH, CH)], osems[sl])
    for sl in range(2):
        if ocp[sl] is not None:
            ocp[sl].wait()


def kernel(x, gate_w, e_score_correction_bias, w_gate_up, w_down,
           ws_gate_up, ws_down):
    T, H = x.shape
    E = gate_w.shape[0]
    FF = w_down.shape[1]
    NTR = (T * TOPK) // M + E
    NR = NTR * M
    NTA = NTR
    NTE = ((NTA // L) + 1) * L

    xb = x.astype(jnp.bfloat16)
    gwb = gate_w.astype(jnp.bfloat16)
    bias16 = jnp.zeros((L,), jnp.float32).at[:E].set(e_score_correction_bias)
    wgu_bf = w_gate_up.astype(jnp.bfloat16)
    wdn_bf = w_down.astype(jnp.bfloat16)
    wsgu_bf = ws_gate_up.astype(jnp.bfloat16)
    wsdn_bf = ws_down.astype(jnp.bfloat16)

    scores = pl.pallas_call(
        _gating_body,
        grid=(1,),
        in_specs=[
            pl.BlockSpec((E, H), lambda i: (0, 0)),
            pl.BlockSpec((T, H), lambda i: (0, 0)),
        ],
        out_specs=pl.BlockSpec((E, T), lambda i: (0, 0)),
        out_shape=jax.ShapeDtypeStruct((E, T), jnp.float32),
    )(gwb, xb)

    mesh = plsc.VectorSubcoreMesh(core_axis_name="c", subcore_axis_name="s")
    rt, te, pos, w01 = pl.kernel(
        functools.partial(_dispatch_body, T=T, E=E, NTR=NTR, NTA=NTA),
        out_type=[
            jax.ShapeDtypeStruct((NR,), jnp.int32),
            jax.ShapeDtypeStruct((NTE,), jnp.int32),
            jax.ShapeDtypeStruct((TOPK, T), jnp.int32),
            jax.ShapeDtypeStruct((TOPK, T), jnp.float32),
        ],
        mesh=mesh,
        compiler_params=pltpu.CompilerParams(needs_layout_passes=False),
        scratch_types=[
            pltpu.VMEM((E, T), jnp.float32),
            pltpu.VMEM((L,), jnp.float32),
            pltpu.VMEM((TOPK, T), jnp.int32),
            pltpu.VMEM((TOPK, T), jnp.float32),
            pltpu.VMEM((TOPK, T), jnp.int32),
            pltpu.VMEM((NR,), jnp.int32),
            pltpu.VMEM((NTE,), jnp.int32),
            pltpu.VMEM((L,), jnp.int32),
            pltpu.VMEM((L,), jnp.int32),
        ],
    )(scores, bias16)

    RPW = NR // 32
    CH = 16
    xs = pl.kernel(
        functools.partial(_gather_body, NR=NR, RPW=RPW, CH=CH),
        out_type=jax.ShapeDtypeStruct((NR, H), jnp.float32),
        mesh=mesh,
        compiler_params=pltpu.CompilerParams(needs_layout_passes=False),
        scratch_types=[
            pltpu.VMEM((RPW,), jnp.int32),
            pltpu.VMEM((CH, H), jnp.float32),
            pltpu.VMEM((CH, H), jnp.float32),
            pltpu.SemaphoreType.DMA,
            pltpu.SemaphoreType.DMA,
            pltpu.SemaphoreType.DMA,
            pltpu.SemaphoreType.DMA,
        ],
    )(x, rt)

    eo = pl.pallas_call(
        functools.partial(_gmm_body, FF=FF),
        grid_spec=pltpu.PrefetchScalarGridSpec(
            num_scalar_prefetch=1,
            grid=(NTA,),
            in_specs=[
                pl.BlockSpec((M, H), lambda j, te_r: (j, 0)),
                pl.BlockSpec((1, H, 2 * FF), lambda j, te_r: (te_r[j], 0, 0)),
                pl.BlockSpec((1, FF, H), lambda j, te_r: (te_r[j], 0, 0)),
            ],
            out_specs=pl.BlockSpec((M, H), lambda j, te_r: (j, 0)),
        ),
        out_shape=jax.ShapeDtypeStruct((NR, H), jnp.float32),
        compiler_params=pltpu.CompilerParams(
            dimension_semantics=("arbitrary",)),
    )(te, xs, wgu_bf, wdn_bf)

    eo_s = pl.pallas_call(
        functools.partial(_shared_body, FF=FF),
        grid=(T // M,),
        in_specs=[
            pl.BlockSpec((M, H), lambda j: (j, 0)),
            pl.BlockSpec((H, 2 * FF), lambda j: (0, 0)),
            pl.BlockSpec((FF, H), lambda j: (0, 0)),
        ],
        out_specs=pl.BlockSpec((M, H), lambda j: (j, 0)),
        out_shape=jax.ShapeDtypeStruct((T, H), jnp.float32),
        compiler_params=pltpu.CompilerParams(
            dimension_semantics=("arbitrary",)),
    )(x, wsgu_bf, wsdn_bf)

    TPW = T // 32
    CHE = 8
    out = pl.kernel(
        functools.partial(_combine_body, T=T, H=H, TPW=TPW, CH=CHE),
        out_type=jax.ShapeDtypeStruct((T, H), jnp.float32),
        mesh=mesh,
        compiler_params=pltpu.CompilerParams(needs_layout_passes=False),
        scratch_types=(
            [pltpu.VMEM((TOPK, TPW), jnp.int32),
             pltpu.VMEM((TOPK, TPW), jnp.float32)]
            + [pltpu.VMEM((CHE, H), jnp.float32) for _ in range(6)]
            + [pltpu.SemaphoreType.DMA for _ in range(8)]
        ),
    )(eo, eo_s, pos, w01)
    return out


# speedup vs baseline: 3.3029x; 1.2564x over previous
import functools

import jax
import jax.numpy as jnp
from jax import lax
from jax.experimental import pallas as pl
from jax.experimental.pallas import tpu as pltpu
from jax.experimental.pallas import tpu_sc as plsc

SCALE = 2.5
TOPK = 2
M = 128
L = 16


def _gating_body(gwb_ref, xb_ref, scores_ref):
    logits = lax.dot_general(
        gwb_ref[...], xb_ref[...], (((1,), (1,)), ((), ())),
        preferred_element_type=jnp.float32)
    scores_ref[...] = 1.0 / (1.0 + jnp.exp(-logits))


def _dispatch_body(scores_hbm, bias_hbm, rt_hbm, te_hbm, pos_hbm, w_hbm,
                   s_v, b_v, i_v, w_v, p_v, rt_v, te_v, cnt_v, woff_v,
                   *, T, E, NTR, NTA):
    wid = lax.axis_index("s") * 2 + lax.axis_index("c")
    NCH = T // L
    SHB = NTR * M

    @pl.when(wid == 0)
    def _():
        pltpu.sync_copy(scores_hbm, s_v)
        pltpu.sync_copy(bias_hbm, b_v)
        lanes = lax.iota(jnp.int32, L)
        ones_i = jnp.ones((L,), jnp.int32)
        cnt_v[...] = jnp.zeros((L,), jnp.int32)

        def init0(c, _):
            rt_v[pl.ds(c * L, L)] = jnp.zeros((L,), jnp.int32)
            return 0
        lax.fori_loop(0, SHB // L, init0, 0)

        bvec = b_v[...]

        def routing(c, _):
            s = [s_v[e, pl.ds(c * L, L)] for e in range(E)]
            b = [s[e] + bvec[e] for e in range(E)]
            m1 = b[0]
            i1 = jnp.zeros((L,), jnp.int32)
            for e in range(1, E):
                gt = b[e] > m1
                m1 = jnp.where(gt, b[e], m1)
                i1 = jnp.where(gt, e, i1)
            m2 = jnp.full((L,), -1e30, jnp.float32)
            i2 = jnp.zeros((L,), jnp.int32)
            for e in range(E):
                ok = jnp.logical_and(i1 != e, b[e] > m2)
                m2 = jnp.where(ok, b[e], m2)
                i2 = jnp.where(ok, e, i2)
            w1 = jnp.zeros((L,), jnp.float32)
            w2 = jnp.zeros((L,), jnp.float32)
            for e in range(E):
                w1 = jnp.where(i1 == e, s[e], w1)
                w2 = jnp.where(i2 == e, s[e], w2)
            den = w1 + w2 + 1e-20
            w1 = w1 / den * SCALE
            w2 = w2 / den * SCALE
            i_v[0, pl.ds(c * L, L)] = i1
            i_v[1, pl.ds(c * L, L)] = i2
            w_v[0, pl.ds(c * L, L)] = w1
            w_v[1, pl.ds(c * L, L)] = w2
            plsc.addupdate_scatter(cnt_v, [i1], ones_i)
            plsc.addupdate_scatter(cnt_v, [i2], ones_i)
            return 0
        lax.fori_loop(0, NCH, routing, 0)

        cnt = cnt_v[...]
        pc = ((cnt + (M - 1)) >> 7) << 7
        cum = plsc.cumsum(pc)
        woff_v[...] = cum - pc
        for jc in range(NTA // L + 1):
            jv = (lanes + jc * L) * M
            tej = jnp.zeros((L,), jnp.int32)
            for e in range(E):
                tej += (jv >= cum[e]).astype(jnp.int32)
            te_v[pl.ds(jc * L, L)] = jnp.minimum(tej, E - 1)

        def place(c, _):
            toks = lanes + c * L
            for k in range(TOPK):
                ev = i_v[k, pl.ds(c * L, L)]
                bases = plsc.load_gather(woff_v, [ev])
                rank = jnp.zeros((L,), jnp.int32)
                for e in range(E):
                    mk = ev == e
                    cs = plsc.cumsum(mk.astype(jnp.int32))
                    rank = jnp.where(mk, cs - 1, rank)
                posv = bases + rank
                plsc.addupdate_scatter(woff_v, [ev], ones_i)
                plsc.store_scatter(rt_v, [posv], toks)
                p_v[k, pl.ds(c * L, L)] = posv
            return 0
        lax.fori_loop(0, NCH, place, 0)

        pltpu.sync_copy(rt_v, rt_hbm)
        pltpu.sync_copy(te_v, te_hbm)
        pltpu.sync_copy(p_v, pos_hbm)
        pltpu.sync_copy(w_v, w_hbm)


def _gather_body(x_hbm, rt_hbm, xs_hbm, idx_v, buf0, buf1, gsem0, gsem1,
                 osem0, osem1, *, NR, RPW, CH):
    wid = lax.axis_index("s") * 2 + lax.axis_index("c")
    base = wid * RPW
    pltpu.sync_copy(rt_hbm.at[pl.ds(base, RPW)], idx_v)
    bufs = (buf0, buf1)
    gsems = (gsem0, gsem1)
    osems = (osem0, osem1)
    nch = RPW // CH
    gcp = [None, None]
    ocp = [None, None]

    def start_fetch(ch, sl):
        gcp[sl] = pltpu.async_copy(
            x_hbm.at[idx_v.at[pl.ds(ch * CH, CH)]], bufs[sl], gsems[sl])

    start_fetch(0, 0)
    for ch in range(nch):
        sl = ch & 1
        if ch + 1 < nch:
            if ocp[1 - sl] is not None:
                ocp[1 - sl].wait()
            start_fetch(ch + 1, 1 - sl)
        gcp[sl].wait()
        ocp[sl] = pltpu.async_copy(
            bufs[sl], xs_hbm.at[pl.ds(base + ch * CH, CH)], osems[sl])
    for sl in range(2):
        if ocp[sl] is not None:
            ocp[sl].wait()


def _gmm_body(te_ref, xs_ref, wgu_ref, wdn_ref, eo_ref, *, FF):
    xb = xs_ref[...]
    gu = lax.dot_general(
        xb, wgu_ref[0], (((1,), (0,)), ((), ())),
        preferred_element_type=jnp.float32)
    g = gu[:, :FF]
    u = gu[:, FF:]
    act = g * (1.0 / (1.0 + jnp.exp(-g))) * u
    eo_ref[...] = lax.dot_general(
        act, wdn_ref[0], (((1,), (0,)), ((), ())),
        preferred_element_type=jnp.float32)


def _shared_body(x_ref, wgu_ref, wdn_ref, eo_ref, *, FF):
    xb = x_ref[...]
    gu = lax.dot_general(
        xb, wgu_ref[...], (((1,), (0,)), ((), ())),
        preferred_element_type=jnp.float32)
    g = gu[:, :FF]
    u = gu[:, FF:]
    act = g * (1.0 / (1.0 + jnp.exp(-g))) * u
    eo_ref[...] = lax.dot_general(
        act, wdn_ref[...], (((1,), (0,)), ((), ())),
        preferred_element_type=jnp.float32)


def _combine_body(eo_hbm, eos_hbm, pos_hbm, w_hbm, out_hbm, p_v, w_v,
                  g0a, g0b, g1a, g1b, sha, shb, sg0a, sg0b, sg1a, sg1b,
                  ssha, sshb, soa, sob, *, T, H, TPW, CH):
    wid = lax.axis_index("s") * 2 + lax.axis_index("c")
    base = wid * TPW
    pltpu.sync_copy(pos_hbm.at[0, pl.ds(base, TPW)], p_v.at[0])
    pltpu.sync_copy(pos_hbm.at[1, pl.ds(base, TPW)], p_v.at[1])
    pltpu.sync_copy(w_hbm.at[0, pl.ds(base, TPW)], w_v.at[0])
    pltpu.sync_copy(w_hbm.at[1, pl.ds(base, TPW)], w_v.at[1])
    nch = TPW // CH
    nv = H // L
    g0s = (g0a, g0b)
    g1s = (g1a, g1b)
    shs = (sha, shb)
    gsems = ((sg0a, sg1a, ssha), (sg0b, sg1b, sshb))
    osems = (soa, sob)
    fcp = [None, None]
    ocp = [None, None]

    def start_fetch(ch, sl):
        s0, s1, s2 = gsems[sl]
        fcp[sl] = (
            pltpu.async_copy(
                eo_hbm.at[p_v.at[0, pl.ds(ch * CH, CH)]], g0s[sl], s0),
            pltpu.async_copy(
                eo_hbm.at[p_v.at[1, pl.ds(ch * CH, CH)]], g1s[sl], s1),
            pltpu.async_copy(
                eos_hbm.at[pl.ds(base + ch * CH, CH)], shs[sl], s2),
        )

    start_fetch(0, 0)
    for ch in range(nch):
        sl = ch & 1
        if ch + 1 < nch:
            if ocp[1 - sl] is not None:
                ocp[1 - sl].wait()
            start_fetch(ch + 1, 1 - sl)
        for cp in fcp[sl]:
            cp.wait()
        g0 = g0s[sl]
        g1 = g1s[sl]
        sh = shs[sl]
        w0vec = w_v[0, pl.ds((ch // 2) * L, L)]
        w1vec = w_v[1, pl.ds((ch // 2) * L, L)]
        for r in range(CH):
            w0 = w0vec[(ch % 2) * CH + r]
            w1 = w1vec[(ch % 2) * CH + r]

            def row(v, _):
                sh[r, pl.ds(v * L, L)] = (
                    sh[r, pl.ds(v * L, L)]
                    + g0[r, pl.ds(v * L, L)] * w0
                    + g1[r, pl.ds(v * L, L)] * w1)
                return 0
            lax.fori_loop(0, nv, row, 0)
        ocp[sl] = pltpu.async_copy(
            sh, out_hbm.at[pl.ds(base + ch * CH, CH)], osems[sl])
    for sl in range(2):
        if ocp[sl] is not None:
            ocp[sl].wait()


def kernel(x, gate_w, e_score_correction_bias, w_gate_up, w_down,
           ws_gate_up, ws_down):
    T, H = x.shape
    E = gate_w.shape[0]
    FF = w_down.shape[1]
    NTR = (T * TOPK) // M + E
    NR = NTR * M
    NTA = NTR
    NTE = ((NTA // L) + 1) * L

    xb = x.astype(jnp.bfloat16)
    gwb = gate_w.astype(jnp.bfloat16)
    bias16 = jnp.zeros((L,), jnp.float32).at[:E].set(e_score_correction_bias)
    wgu_bf = w_gate_up
    wdn_bf = w_down
    wsgu_bf = ws_gate_up
    wsdn_bf = ws_down

    scores = pl.pallas_call(
        _gating_body,
        grid=(1,),
        in_specs=[
            pl.BlockSpec((E, H), lambda i: (0, 0)),
            pl.BlockSpec((T, H), lambda i: (0, 0)),
        ],
        out_specs=pl.BlockSpec((E, T), lambda i: (0, 0)),
        out_shape=jax.ShapeDtypeStruct((E, T), jnp.float32),
    )(gwb, xb)

    mesh = plsc.VectorSubcoreMesh(core_axis_name="c", subcore_axis_name="s")
    rt, te, pos, w01 = pl.kernel(
        functools.partial(_dispatch_body, T=T, E=E, NTR=NTR, NTA=NTA),
        out_type=[
            jax.ShapeDtypeStruct((NR,), jnp.int32),
            jax.ShapeDtypeStruct((NTE,), jnp.int32),
            jax.ShapeDtypeStruct((TOPK, T), jnp.int32),
            jax.ShapeDtypeStruct((TOPK, T), jnp.float32),
        ],
        mesh=mesh,
        compiler_params=pltpu.CompilerParams(needs_layout_passes=False),
        scratch_types=[
            pltpu.VMEM((E, T), jnp.float32),
            pltpu.VMEM((L,), jnp.float32),
            pltpu.VMEM((TOPK, T), jnp.int32),
            pltpu.VMEM((TOPK, T), jnp.float32),
            pltpu.VMEM((TOPK, T), jnp.int32),
            pltpu.VMEM((NR,), jnp.int32),
            pltpu.VMEM((NTE,), jnp.int32),
            pltpu.VMEM((L,), jnp.int32),
            pltpu.VMEM((L,), jnp.int32),
        ],
    )(scores, bias16)

    RPW = NR // 32
    CH = 16
    xs = pl.kernel(
        functools.partial(_gather_body, NR=NR, RPW=RPW, CH=CH),
        out_type=jax.ShapeDtypeStruct((NR, H), jnp.float32),
        mesh=mesh,
        compiler_params=pltpu.CompilerParams(needs_layout_passes=False),
        scratch_types=[
            pltpu.VMEM((RPW,), jnp.int32),
            pltpu.VMEM((CH, H), jnp.float32),
            pltpu.VMEM((CH, H), jnp.float32),
            pltpu.SemaphoreType.DMA,
            pltpu.SemaphoreType.DMA,
            pltpu.SemaphoreType.DMA,
            pltpu.SemaphoreType.DMA,
        ],
    )(x, rt)

    eo = pl.pallas_call(
        functools.partial(_gmm_body, FF=FF),
        grid_spec=pltpu.PrefetchScalarGridSpec(
            num_scalar_prefetch=1,
            grid=(NTA,),
            in_specs=[
                pl.BlockSpec((M, H), lambda j, te_r: (j, 0)),
                pl.BlockSpec((1, H, 2 * FF), lambda j, te_r: (te_r[j], 0, 0)),
                pl.BlockSpec((1, FF, H), lambda j, te_r: (te_r[j], 0, 0)),
            ],
            out_specs=pl.BlockSpec((M, H), lambda j, te_r: (j, 0)),
        ),
        out_shape=jax.ShapeDtypeStruct((NR, H), jnp.float32),
        compiler_params=pltpu.CompilerParams(
            dimension_semantics=("arbitrary",)),
    )(te, xs, wgu_bf, wdn_bf)

    eo_s = pl.pallas_call(
        functools.partial(_shared_body, FF=FF),
        grid=(T // M,),
        in_specs=[
            pl.BlockSpec((M, H), lambda j: (j, 0)),
            pl.BlockSpec((H, 2 * FF), lambda j: (0, 0)),
            pl.BlockSpec((FF, H), lambda j: (0, 0)),
        ],
        out_specs=pl.BlockSpec((M, H), lambda j: (j, 0)),
        out_shape=jax.ShapeDtypeStruct((T, H), jnp.float32),
        compiler_params=pltpu.CompilerParams(
            dimension_semantics=("arbitrary",)),
    )(x, wsgu_bf, wsdn_bf)

    TPW = T // 32
    CHE = 8
    out = pl.kernel(
        functools.partial(_combine_body, T=T, H=H, TPW=TPW, CH=CHE),
        out_type=jax.ShapeDtypeStruct((T, H), jnp.float32),
        mesh=mesh,
        compiler_params=pltpu.CompilerParams(needs_layout_passes=False),
        scratch_types=(
            [pltpu.VMEM((TOPK, TPW), jnp.int32),
             pltpu.VMEM((TOPK, TPW), jnp.float32)]
            + [pltpu.VMEM((CHE, H), jnp.float32) for _ in range(6)]
            + [pltpu.SemaphoreType.DMA for _ in range(8)]
        ),
    )(eo, eo_s, pos, w01)
    return out


# speedup vs baseline: 3.4039x; 1.0306x over previous
import functools

import jax
import jax.numpy as jnp
from jax import lax
from jax.experimental import pallas as pl
from jax.experimental.pallas import tpu as pltpu
from jax.experimental.pallas import tpu_sc as plsc

SCALE = 2.5
TOPK = 2
M = 128
L = 16


def _gating_body(gwb_ref, xb_ref, scores_ref):
    logits = lax.dot_general(
        gwb_ref[...], xb_ref[...], (((1,), (1,)), ((), ())),
        preferred_element_type=jnp.float32)
    scores_ref[...] = 1.0 / (1.0 + jnp.exp(-logits))


def _dispatch_body(scores_hbm, bias_hbm, rt_hbm, te_hbm, pos_hbm, w_hbm,
                   s_v, b_v, i_v, w_v, p_v, rt_v, te_v, cnt_v, woff_v,
                   *, T, E, NTR, NTA):
    wid = lax.axis_index("s") * 2 + lax.axis_index("c")
    NCH = T // L
    SHB = NTR * M

    @pl.when(wid == 0)
    def _():
        pltpu.sync_copy(scores_hbm, s_v)
        pltpu.sync_copy(bias_hbm, b_v)
        lanes = lax.iota(jnp.int32, L)
        ones_i = jnp.ones((L,), jnp.int32)
        cnt_v[...] = jnp.zeros((L,), jnp.int32)

        def init0(c, _):
            rt_v[pl.ds(c * L, L)] = jnp.zeros((L,), jnp.int32)
            return 0
        lax.fori_loop(0, SHB // L, init0, 0)

        bvec = b_v[...]

        def routing(c, _):
            s = [s_v[e, pl.ds(c * L, L)] for e in range(E)]
            b = [s[e] + bvec[e] for e in range(E)]
            m1 = b[0]
            i1 = jnp.zeros((L,), jnp.int32)
            for e in range(1, E):
                gt = b[e] > m1
                m1 = jnp.where(gt, b[e], m1)
                i1 = jnp.where(gt, e, i1)
            m2 = jnp.full((L,), -1e30, jnp.float32)
            i2 = jnp.zeros((L,), jnp.int32)
            for e in range(E):
                ok = jnp.logical_and(i1 != e, b[e] > m2)
                m2 = jnp.where(ok, b[e], m2)
                i2 = jnp.where(ok, e, i2)
            w1 = jnp.zeros((L,), jnp.float32)
            w2 = jnp.zeros((L,), jnp.float32)
            for e in range(E):
                w1 = jnp.where(i1 == e, s[e], w1)
                w2 = jnp.where(i2 == e, s[e], w2)
            den = w1 + w2 + 1e-20
            w1 = w1 / den * SCALE
            w2 = w2 / den * SCALE
            i_v[0, pl.ds(c * L, L)] = i1
            i_v[1, pl.ds(c * L, L)] = i2
            w_v[0, pl.ds(c * L, L)] = w1
            w_v[1, pl.ds(c * L, L)] = w2
            plsc.addupdate_scatter(cnt_v, [i1], ones_i)
            plsc.addupdate_scatter(cnt_v, [i2], ones_i)
            return 0
        lax.fori_loop(0, NCH, routing, 0)

        cnt = cnt_v[...]
        pc = ((cnt + (M - 1)) >> 7) << 7
        cum = plsc.cumsum(pc)
        woff_v[...] = cum - pc
        for jc in range(NTA // L + 1):
            jv = (lanes + jc * L) * M
            tej = jnp.zeros((L,), jnp.int32)
            for e in range(E):
                tej += (jv >= cum[e]).astype(jnp.int32)
            te_v[pl.ds(jc * L, L)] = jnp.minimum(tej, E - 1)

        def place(c, _):
            toks = lanes + c * L
            for k in range(TOPK):
                ev = i_v[k, pl.ds(c * L, L)]
                bases = plsc.load_gather(woff_v, [ev])
                rank = jnp.zeros((L,), jnp.int32)
                for e in range(E):
                    mk = ev == e
                    cs = plsc.cumsum(mk.astype(jnp.int32))
                    rank = jnp.where(mk, cs - 1, rank)
                posv = bases + rank
                plsc.addupdate_scatter(woff_v, [ev], ones_i)
                plsc.store_scatter(rt_v, [posv], toks)
                p_v[k, pl.ds(c * L, L)] = posv
            return 0
        lax.fori_loop(0, NCH, place, 0)

        pltpu.sync_copy(rt_v, rt_hbm)
        pltpu.sync_copy(te_v, te_hbm)
        pltpu.sync_copy(p_v, pos_hbm)
        pltpu.sync_copy(w_v, w_hbm)


def _gather_body(x_hbm, rt_hbm, xs_hbm, idx_v, buf0, buf1, gsem0, gsem1,
                 osem0, osem1, *, NR, RPW, CH):
    wid = lax.axis_index("s") * 2 + lax.axis_index("c")
    base = wid * RPW
    pltpu.sync_copy(rt_hbm.at[pl.ds(base, RPW)], idx_v)
    bufs = (buf0, buf1)
    gsems = (gsem0, gsem1)
    osems = (osem0, osem1)
    nch = RPW // CH
    gcp = [None, None]
    ocp = [None, None]

    def start_fetch(ch, sl):
        gcp[sl] = pltpu.async_copy(
            x_hbm.at[idx_v.at[pl.ds(ch * CH, CH)]], bufs[sl], gsems[sl])

    start_fetch(0, 0)
    for ch in range(nch):
        sl = ch & 1
        if ch + 1 < nch:
            if ocp[1 - sl] is not None:
                ocp[1 - sl].wait()
            start_fetch(ch + 1, 1 - sl)
        gcp[sl].wait()
        ocp[sl] = pltpu.async_copy(
            bufs[sl], xs_hbm.at[pl.ds(base + ch * CH, CH)], osems[sl])
    for sl in range(2):
        if ocp[sl] is not None:
            ocp[sl].wait()


def _gmm_body(te_ref, xs_ref, wgu_ref, wdn_ref, eo_ref, *, FF):
    xb = xs_ref[...]
    gu = lax.dot_general(
        xb, wgu_ref[0], (((1,), (0,)), ((), ())),
        preferred_element_type=jnp.float32)
    g = gu[:, :FF]
    u = gu[:, FF:]
    act = g * (1.0 / (1.0 + jnp.exp(-g))) * u
    eo_ref[...] = lax.dot_general(
        act, wdn_ref[0], (((1,), (0,)), ((), ())),
        preferred_element_type=jnp.float32)


def _shared_body(x_ref, wgu_ref, wdn_ref, eo_ref, *, FF):
    xb = x_ref[...]
    gu = lax.dot_general(
        xb, wgu_ref[...], (((1,), (0,)), ((), ())),
        preferred_element_type=jnp.float32)
    g = gu[:, :FF]
    u = gu[:, FF:]
    act = g * (1.0 / (1.0 + jnp.exp(-g))) * u
    eo_ref[...] = lax.dot_general(
        act, wdn_ref[...], (((1,), (0,)), ((), ())),
        preferred_element_type=jnp.float32)


def _combine_body(eo_hbm, eos_hbm, pos_hbm, w_hbm, out_hbm, p_v, w_v,
                  g0a, g0b, g1a, g1b, sha, shb, sg0a, sg0b, sg1a, sg1b,
                  ssha, sshb, soa, sob, *, T, H, TPW, CH):
    wid = lax.axis_index("s") * 2 + lax.axis_index("c")
    base = wid * TPW
    pltpu.sync_copy(pos_hbm.at[0, pl.ds(base, TPW)], p_v.at[0])
    pltpu.sync_copy(pos_hbm.at[1, pl.ds(base, TPW)], p_v.at[1])
    pltpu.sync_copy(w_hbm.at[0, pl.ds(base, TPW)], w_v.at[0])
    pltpu.sync_copy(w_hbm.at[1, pl.ds(base, TPW)], w_v.at[1])
    nch = TPW // CH
    nv = H // L
    g0s = (g0a, g0b)
    g1s = (g1a, g1b)
    shs = (sha, shb)
    gsems = ((sg0a, sg1a, ssha), (sg0b, sg1b, sshb))
    osems = (soa, sob)
    fcp = [None, None]
    ocp = [None, None]

    def start_fetch(ch, sl):
        s0, s1, s2 = gsems[sl]
        fcp[sl] = (
            pltpu.async_copy(
                eo_hbm.at[p_v.at[0, pl.ds(ch * CH, CH)]], g0s[sl], s0),
            pltpu.async_copy(
                eo_hbm.at[p_v.at[1, pl.ds(ch * CH, CH)]], g1s[sl], s1),
            pltpu.async_copy(
                eos_hbm.at[pl.ds(base + ch * CH, CH)], shs[sl], s2),
        )

    start_fetch(0, 0)
    for ch in range(nch):
        sl = ch & 1
        if ch + 1 < nch:
            if ocp[1 - sl] is not None:
                ocp[1 - sl].wait()
            start_fetch(ch + 1, 1 - sl)
        for cp in fcp[sl]:
            cp.wait()
        g0 = g0s[sl]
        g1 = g1s[sl]
        sh = shs[sl]
        w0vec = w_v[0, pl.ds((ch // 2) * L, L)]
        w1vec = w_v[1, pl.ds((ch // 2) * L, L)]
        for r in range(CH):
            w0 = w0vec[(ch % 2) * CH + r]
            w1 = w1vec[(ch % 2) * CH + r]

            def row(v, _):
                for q in range(4):
                    o = v * (4 * L) + q * L
                    sh[r, pl.ds(o, L)] = (
                        sh[r, pl.ds(o, L)]
                        + g0[r, pl.ds(o, L)] * w0
                        + g1[r, pl.ds(o, L)] * w1)
                return 0
            lax.fori_loop(0, nv // 4, row, 0)
        ocp[sl] = pltpu.async_copy(
            sh, out_hbm.at[pl.ds(base + ch * CH, CH)], osems[sl])
    for sl in range(2):
        if ocp[sl] is not None:
            ocp[sl].wait()


def kernel(x, gate_w, e_score_correction_bias, w_gate_up, w_down,
           ws_gate_up, ws_down):
    T, H = x.shape
    E = gate_w.shape[0]
    FF = w_down.shape[1]
    NTR = (T * TOPK) // M + E
    NR = NTR * M
    NTA = NTR
    NTE = ((NTA // L) + 1) * L

    xb = x.astype(jnp.bfloat16)
    gwb = gate_w.astype(jnp.bfloat16)
    bias16 = jnp.zeros((L,), jnp.float32).at[:E].set(e_score_correction_bias)
    wgu_bf = w_gate_up
    wdn_bf = w_down
    wsgu_bf = ws_gate_up
    wsdn_bf = ws_down

    scores = pl.pallas_call(
        _gating_body,
        grid=(1,),
        in_specs=[
            pl.BlockSpec((E, H), lambda i: (0, 0)),
            pl.BlockSpec((T, H), lambda i: (0, 0)),
        ],
        out_specs=pl.BlockSpec((E, T), lambda i: (0, 0)),
        out_shape=jax.ShapeDtypeStruct((E, T), jnp.float32),
    )(gwb, xb)

    mesh = plsc.VectorSubcoreMesh(core_axis_name="c", subcore_axis_name="s")
    rt, te, pos, w01 = pl.kernel(
        functools.partial(_dispatch_body, T=T, E=E, NTR=NTR, NTA=NTA),
        out_type=[
            jax.ShapeDtypeStruct((NR,), jnp.int32),
            jax.ShapeDtypeStruct((NTE,), jnp.int32),
            jax.ShapeDtypeStruct((TOPK, T), jnp.int32),
            jax.ShapeDtypeStruct((TOPK, T), jnp.float32),
        ],
        mesh=mesh,
        compiler_params=pltpu.CompilerParams(needs_layout_passes=False),
        scratch_types=[
            pltpu.VMEM((E, T), jnp.float32),
            pltpu.VMEM((L,), jnp.float32),
            pltpu.VMEM((TOPK, T), jnp.int32),
            pltpu.VMEM((TOPK, T), jnp.float32),
            pltpu.VMEM((TOPK, T), jnp.int32),
            pltpu.VMEM((NR,), jnp.int32),
            pltpu.VMEM((NTE,), jnp.int32),
            pltpu.VMEM((L,), jnp.int32),
            pltpu.VMEM((L,), jnp.int32),
        ],
    )(scores, bias16)

    RPW = NR // 32
    CH = 16
    xs = pl.kernel(
        functools.partial(_gather_body, NR=NR, RPW=RPW, CH=CH),
        out_type=jax.ShapeDtypeStruct((NR, H), jnp.float32),
        mesh=mesh,
        compiler_params=pltpu.CompilerParams(needs_layout_passes=False),
        scratch_types=[
            pltpu.VMEM((RPW,), jnp.int32),
            pltpu.VMEM((CH, H), jnp.float32),
            pltpu.VMEM((CH, H), jnp.float32),
            pltpu.SemaphoreType.DMA,
            pltpu.SemaphoreType.DMA,
            pltpu.SemaphoreType.DMA,
            pltpu.SemaphoreType.DMA,
        ],
    )(x, rt)

    eo = pl.pallas_call(
        functools.partial(_gmm_body, FF=FF),
        grid_spec=pltpu.PrefetchScalarGridSpec(
            num_scalar_prefetch=1,
            grid=(NTA,),
            in_specs=[
                pl.BlockSpec((M, H), lambda j, te_r: (j, 0)),
                pl.BlockSpec((1, H, 2 * FF), lambda j, te_r: (te_r[j], 0, 0)),
                pl.BlockSpec((1, FF, H), lambda j, te_r: (te_r[j], 0, 0)),
            ],
            out_specs=pl.BlockSpec((M, H), lambda j, te_r: (j, 0)),
        ),
        out_shape=jax.ShapeDtypeStruct((NR, H), jnp.float32),
        compiler_params=pltpu.CompilerParams(
            dimension_semantics=("arbitrary",)),
    )(te, xs, wgu_bf, wdn_bf)

    eo_s = pl.pallas_call(
        functools.partial(_shared_body, FF=FF),
        grid=(T // M,),
        in_specs=[
            pl.BlockSpec((M, H), lambda j: (j, 0)),
            pl.BlockSpec((H, 2 * FF), lambda j: (0, 0)),
            pl.BlockSpec((FF, H), lambda j: (0, 0)),
        ],
        out_specs=pl.BlockSpec((M, H), lambda j: (j, 0)),
        out_shape=jax.ShapeDtypeStruct((T, H), jnp.float32),
        compiler_params=pltpu.CompilerParams(
            dimension_semantics=("arbitrary",)),
    )(x, wsgu_bf, wsdn_bf)

    TPW = T // 32
    CHE = 8
    out = pl.kernel(
        functools.partial(_combine_body, T=T, H=H, TPW=TPW, CH=CHE),
        out_type=jax.ShapeDtypeStruct((T, H), jnp.float32),
        mesh=mesh,
        compiler_params=pltpu.CompilerParams(needs_layout_passes=False),
        scratch_types=(
            [pltpu.VMEM((TOPK, TPW), jnp.int32),
             pltpu.VMEM((TOPK, TPW), jnp.float32)]
            + [pltpu.VMEM((CHE, H), jnp.float32) for _ in range(6)]
            + [pltpu.SemaphoreType.DMA for _ in range(8)]
        ),
    )(eo, eo_s, pos, w01)
    return out
